# baseline (speedup 1.0000x reference)
import functools

import jax
import jax.numpy as jnp
from jax import lax
from jax.experimental import pallas as pl
from jax.experimental.pallas import tpu as pltpu
from jax.experimental.pallas import tpu_sc as plsc

B = 16384
D = 128
K = 30
EPS_NORM = 1e-12

NW = 32
ROWS_PW = B // NW
IDX_PW = ROWS_PW * K
CHUNK_ROWS = 8
CHUNK_IDX = CHUNK_ROWS * K
HALF_IDX = CHUNK_IDX // 2
NCHUNKS = IDX_PW // CHUNK_IDX
ROWS_SUPER = 64
CHUNKS_PER_SUPER = ROWS_SUPER // CHUNK_ROWS
LANES = 16
NCHUNK16 = D // LANES

BB = 1024


def _normalize_body(zi_ref, zj_ref, out_ref, outpk_ref, pos_ref):
    ys = []
    for s, ref in ((0, zi_ref), (1, zj_ref)):
        x = ref[...]
        n = jnp.sqrt(jnp.sum(x * x, axis=1, keepdims=True))
        y = x / jnp.maximum(n, EPS_NORM)
        ys.append(y)
        out_ref[s] = y
        yb = lax.bitcast_convert_type(y, jnp.int32)
        r = yb + jnp.int32(0x7FFF) + ((yb >> 16) & 1)
        a = r[:, : D // 2]
        b = r[:, D // 2 :]
        w = ((a >> 16) & jnp.int32(0xFFFF)) | (b & jnp.int32(-65536))
        outpk_ref[s] = w
    pos_ref[...] = jnp.sum(ys[0] * ys[1], axis=1, keepdims=True)


_normalize = pl.pallas_call(
    _normalize_body,
    grid=(B // BB,),
    in_specs=[
        pl.BlockSpec((BB, D), lambda i: (i, 0)),
        pl.BlockSpec((BB, D), lambda i: (i, 0)),
    ],
    out_specs=[
        pl.BlockSpec((2, BB, D), lambda i: (0, i, 0)),
        pl.BlockSpec((2, BB, D // 2), lambda i: (0, i, 0)),
        pl.BlockSpec((BB, 1), lambda i: (i, 0)),
    ],
    out_shape=[
        jax.ShapeDtypeStruct((2, B, D), jnp.float32),
        jax.ShapeDtypeStruct((2, B, D // 2), jnp.int32),
        jax.ShapeDtypeStruct((B, 1), jnp.float32),
    ],
)


_GATHER_1D = lax.GatherDimensionNumbers(
    offset_dims=(), collapsed_slice_dims=(0,), start_index_map=(0,)
)


def _lane_perm(x, perm_2d):
    return lax.gather(
        x,
        perm_2d,
        dimension_numbers=_GATHER_1D,
        slice_sizes=(1,),
        mode=lax.GatherScatterMode.PROMISE_IN_BOUNDS,
    )


def _lane_sum_16(accs, perms, uppers):
    vecs = list(accs)
    for lev in range(4):
        perm = perms[lev]
        upper = uppers[lev]
        nxt = []
        for p in range(0, len(vecs), 2):
            ta = vecs[p] + _lane_perm(vecs[p], perm)
            tb = vecs[p + 1] + _lane_perm(vecs[p + 1], perm)
            nxt.append(jnp.where(upper, tb, ta))
        vecs = nxt
    return vecs[0]


def _sc_negdot_body(
    pool_hbm, poolpk_hbm, idx_hbm, out_hbm, idx_v, zi_v, rows_v, out_v,
    sem0, sem1
):
    wid = lax.axis_index("s") * 2 + lax.axis_index("c")
    idx_base = wid * IDX_PW
    row_base = wid * ROWS_PW
    lane_iota = lax.iota(jnp.int32, LANES)
    perms = [(lane_iota ^ (1 << lev))[:, None] for lev in range(4)]
    uppers = [(lane_iota & (1 << lev)) != 0 for lev in range(4)]
    pltpu.sync_copy(idx_hbm.at[pl.ds(idx_base, IDX_PW)], idx_v)

    def fire(c):
        def srcs(h):
            return poolpk_hbm.at[
                idx_v.at[pl.ds(c * CHUNK_IDX + h * HALF_IDX, HALF_IDX)]
            ]

        @pl.when((c & 1) == 0)
        def _():
            pltpu.async_copy(srcs(0), rows_v.at[0, pl.ds(0, HALF_IDX)], sem0)
            pltpu.async_copy(srcs(1), rows_v.at[0, pl.ds(HALF_IDX, HALF_IDX)], sem0)

        @pl.when((c & 1) == 1)
        def _():
            pltpu.async_copy(srcs(0), rows_v.at[1, pl.ds(0, HALF_IDX)], sem1)
            pltpu.async_copy(srcs(1), rows_v.at[1, pl.ds(HALF_IDX, HALF_IDX)], sem1)

    fire(0)

    def chunk_body(cc, carry):
        @pl.when(lax.rem(cc, CHUNKS_PER_SUPER) == 0)
        def _():
            pltpu.sync_copy(
                pool_hbm.at[
                    pl.ds(row_base + (cc // CHUNKS_PER_SUPER) * ROWS_SUPER,
                          ROWS_SUPER)
                ],
                zi_v,
            )

        @pl.when(cc + 1 < NCHUNKS)
        def _():
            fire(cc + 1)

        dummy = poolpk_hbm.at[pl.ds(0, HALF_IDX)]

        @pl.when((cc & 1) == 0)
        def _():
            pltpu.make_async_copy(dummy, rows_v.at[0, pl.ds(0, HALF_IDX)], sem0).wait()
            pltpu.make_async_copy(dummy, rows_v.at[0, pl.ds(0, HALF_IDX)], sem0).wait()

        @pl.when((cc & 1) == 1)
        def _():
            pltpu.make_async_copy(dummy, rows_v.at[1, pl.ds(0, HALF_IDX)], sem1).wait()
            pltpu.make_async_copy(dummy, rows_v.at[1, pl.ds(0, HALF_IDX)], sem1).wait()

        par = cc & 1
        off = cc * CHUNK_IDX
        rloc = lax.rem(cc, CHUNKS_PER_SUPER) * CHUNK_ROWS
        zi_regs = None
        for grp in range(CHUNK_IDX // LANES):
            accs = []
            for l in range(LANES):
                g = grp * LANES + l
                if g % K == 0:
                    zi_regs = [
                        zi_v[rloc + g // K, pl.ds(LANES * t, LANES)]
                        for t in range(NCHUNK16)
                    ]
                acc = None
                for t2 in range(D // 32):
                    w = rows_v[par, g, pl.ds(LANES * t2, LANES)]
                    e = lax.bitcast_convert_type(w << 16, jnp.float32)
                    o = lax.bitcast_convert_type(w, jnp.float32)
                    pa = e * zi_regs[t2]
                    pb = o * zi_regs[t2 + NCHUNK16 // 2]
                    acc = pa + pb if acc is None else acc + pa + pb
                accs.append(acc)
            dots = _lane_sum_16(accs, perms, uppers)
            out_v[pl.ds(off + grp * LANES, LANES)] = dots
        return carry

    lax.fori_loop(0, NCHUNKS, chunk_body, 0)
    pltpu.sync_copy(out_v, out_hbm.at[pl.ds(idx_base, IDX_PW)])


@functools.cache
def _get_sc_negdot():
    return pl.kernel(
        _sc_negdot_body,
        out_type=jax.ShapeDtypeStruct((B * K,), jnp.float32),
        mesh=plsc.VectorSubcoreMesh(core_axis_name="c", subcore_axis_name="s"),
        compiler_params=pltpu.CompilerParams(use_tc_tiling_on_sc=False),
        scratch_types=[
            pltpu.VMEM((IDX_PW,), jnp.int32),
            pltpu.VMEM((ROWS_SUPER, D), jnp.float32),
            pltpu.VMEM((2, CHUNK_IDX, D // 2), jnp.int32),
            pltpu.VMEM((IDX_PW,), jnp.float32),
            pltpu.SemaphoreType.DMA,
            pltpu.SemaphoreType.DMA,
        ],
    )


def _loss_body(temp_ref, pos_ref, neg_ref, out_ref):
    inv_t = 1.0 / temp_ref[0]
    posl = pos_ref[...] * inv_t
    x = neg_ref[...] * inv_t
    m = jnp.maximum(jnp.max(x, axis=1, keepdims=True), posl)
    s = jnp.sum(jnp.exp(x - m), axis=1) + jnp.exp(posl - m)[:, 0]
    lse = m[:, 0] + jnp.log(s)
    contrib = jnp.sum(lse - posl[:, 0]) * (1.0 / B)
    out_ref[...] = jnp.full((1, 1), contrib, dtype=jnp.float32)


_loss = pl.pallas_call(
    _loss_body,
    grid=(1,),
    in_specs=[
        pl.BlockSpec(memory_space=pltpu.SMEM),
        pl.BlockSpec((B, 1), lambda i: (0, 0)),
        pl.BlockSpec((B, K), lambda i: (0, 0)),
    ],
    out_specs=pl.BlockSpec((1, 1), lambda i: (0, 0)),
    out_shape=jax.ShapeDtypeStruct((1, 1), jnp.float32),
)


def kernel(z_i, z_j, temperature, neg_indices):
    pool3, poolpk3, pos = _normalize(z_i, z_j)
    pool = pool3.reshape(2 * B, D)
    poolpk = poolpk3.reshape(2 * B, D // 2)
    neg_flat = _get_sc_negdot()(pool, poolpk, neg_indices.reshape(B * K))
    neg = neg_flat.reshape(B, K)
    out = _loss(temperature.reshape(1), pos, neg)
    return out[0, 0]


# speedup vs baseline: 1.4936x; 1.4936x over previous
import functools

import jax
import jax.numpy as jnp
from jax import lax
from jax.experimental import pallas as pl
from jax.experimental.pallas import tpu as pltpu
from jax.experimental.pallas import tpu_sc as plsc

B = 16384
D = 128
K = 30
EPS_NORM = 1e-12

NW = 32
ROWS_PW = B // NW
IDX_PW = ROWS_PW * K
ROWS_SUPER = 64
CHUNK_ROWS = 4
CHUNK_STRIDE = CHUNK_ROWS * K
CHUNK_IDX = 128
CHUNKS_PER_SUPER = ROWS_SUPER // CHUNK_ROWS
IDX_PAD = IDX_PW + 16
LANES = 16
NCHUNK16 = D // LANES

BB = 1024


def _normalize_body(zi_ref, zj_ref, out_ref, outpk_ref, pos_ref):
    ys = []
    for s, ref in ((0, zi_ref), (1, zj_ref)):
        x = ref[...]
        n = jnp.sqrt(jnp.sum(x * x, axis=1, keepdims=True))
        y = x / jnp.maximum(n, EPS_NORM)
        ys.append(y)
        out_ref[s] = y
        yb = lax.bitcast_convert_type(y, jnp.int32)
        r = yb + jnp.int32(0x7FFF) + ((yb >> 16) & 1)
        a = r[:, : D // 2]
        b = r[:, D // 2 :]
        w = ((a >> 16) & jnp.int32(0xFFFF)) | (b & jnp.int32(-65536))
        outpk_ref[s] = w
    pos_ref[...] = jnp.sum(ys[0] * ys[1], axis=1, keepdims=True)


_normalize = pl.pallas_call(
    _normalize_body,
    grid=(B // BB,),
    in_specs=[
        pl.BlockSpec((BB, D), lambda i: (i, 0)),
        pl.BlockSpec((BB, D), lambda i: (i, 0)),
    ],
    out_specs=[
        pl.BlockSpec((2, BB, D), lambda i: (0, i, 0)),
        pl.BlockSpec((2, BB, D // 2), lambda i: (0, i, 0)),
        pl.BlockSpec((BB, 1), lambda i: (i, 0)),
    ],
    out_shape=[
        jax.ShapeDtypeStruct((2, B, D), jnp.float32),
        jax.ShapeDtypeStruct((2, B, D // 2), jnp.int32),
        jax.ShapeDtypeStruct((B, 1), jnp.float32),
    ],
)


_GATHER_1D = lax.GatherDimensionNumbers(
    offset_dims=(), collapsed_slice_dims=(0,), start_index_map=(0,)
)


def _lane_perm(x, perm_2d):
    return lax.gather(
        x,
        perm_2d,
        dimension_numbers=_GATHER_1D,
        slice_sizes=(1,),
        mode=lax.GatherScatterMode.PROMISE_IN_BOUNDS,
    )


def _lane_sum_16(accs, perms, uppers):
    vecs = list(accs)
    for lev in range(4):
        perm = perms[lev]
        upper = uppers[lev]
        nxt = []
        for p in range(0, len(vecs), 2):
            ta = vecs[p] + _lane_perm(vecs[p], perm)
            tb = vecs[p + 1] + _lane_perm(vecs[p + 1], perm)
            nxt.append(jnp.where(upper, tb, ta))
        vecs = nxt
    return vecs[0]


NCHUNKS = IDX_PW // CHUNK_STRIDE


def _sc_negdot_body(
    pool_hbm, poolpk_hbm, idx_hbm, out_hbm, idx_v, zi_v, rows_v, out_v,
    sem0, sem1
):
    wid = lax.axis_index("s") * 2 + lax.axis_index("c")
    idx_base = wid * IDX_PW
    row_base = wid * ROWS_PW
    lane_iota = lax.iota(jnp.int32, LANES)
    perms = [(lane_iota ^ (1 << lev))[:, None] for lev in range(4)]
    uppers = [(lane_iota & (1 << lev)) != 0 for lev in range(4)]
    pltpu.sync_copy(idx_hbm.at[pl.ds(idx_base, IDX_PW)], idx_v.at[pl.ds(0, IDX_PW)])
    idx_v[pl.ds(IDX_PW, LANES)] = jnp.zeros((LANES,), jnp.int32)

    def fire(c):
        src = poolpk_hbm.at[idx_v.at[pl.ds(c * CHUNK_STRIDE, CHUNK_IDX)]]

        @pl.when((c & 1) == 0)
        def _():
            pltpu.async_copy(src, rows_v.at[0], sem0)

        @pl.when((c & 1) == 1)
        def _():
            pltpu.async_copy(src, rows_v.at[1], sem1)

    fire(0)

    def chunk_body(cc, carry):
        @pl.when(lax.rem(cc, CHUNKS_PER_SUPER) == 0)
        def _():
            pltpu.sync_copy(
                pool_hbm.at[
                    pl.ds(row_base + (cc // CHUNKS_PER_SUPER) * ROWS_SUPER,
                          ROWS_SUPER + 1)
                ],
                zi_v,
            )

        @pl.when(cc + 1 < NCHUNKS)
        def _():
            fire(cc + 1)

        dummy = poolpk_hbm.at[pl.ds(0, CHUNK_IDX)]

        @pl.when((cc & 1) == 0)
        def _():
            pltpu.make_async_copy(dummy, rows_v.at[0], sem0).wait()

        @pl.when((cc & 1) == 1)
        def _():
            pltpu.make_async_copy(dummy, rows_v.at[1], sem1).wait()

        par = cc & 1
        off = cc * CHUNK_STRIDE
        rloc = lax.rem(cc, CHUNKS_PER_SUPER) * CHUNK_ROWS
        zi_regs = None
        for grp in range(CHUNK_IDX // LANES):
            if True:
                accs = []
                for l in range(LANES):
                    g = grp * LANES + l
                    if g % K == 0:
                        zi_regs = [
                            zi_v[rloc + g // K, pl.ds(LANES * t, LANES)]
                            for t in range(NCHUNK16)
                        ]
                    acc = None
                    for t2 in range(D // 32):
                        w = rows_v[par, g, pl.ds(LANES * t2, LANES)]
                        e = lax.bitcast_convert_type(w << 16, jnp.float32)
                        o = lax.bitcast_convert_type(w, jnp.float32)
                        pa = e * zi_regs[t2]
                        pb = o * zi_regs[t2 + NCHUNK16 // 2]
                        acc = pa + pb if acc is None else acc + pa + pb
                    accs.append(acc)
                dots = _lane_sum_16(accs, perms, uppers)
                out_v[pl.ds(off + grp * LANES, LANES)] = dots
        return carry

    lax.fori_loop(0, NCHUNKS, chunk_body, 0)
    pltpu.sync_copy(out_v.at[pl.ds(0, IDX_PW)], out_hbm.at[pl.ds(idx_base, IDX_PW)])


@functools.cache
def _get_sc_negdot():
    return pl.kernel(
        _sc_negdot_body,
        out_type=jax.ShapeDtypeStruct((B * K,), jnp.float32),
        mesh=plsc.VectorSubcoreMesh(core_axis_name="c", subcore_axis_name="s"),
        compiler_params=pltpu.CompilerParams(use_tc_tiling_on_sc=False),
        scratch_types=[
            pltpu.VMEM((IDX_PAD,), jnp.int32),
            pltpu.VMEM((ROWS_SUPER + 1, D), jnp.float32),
            pltpu.VMEM((2, CHUNK_IDX, D // 2), jnp.int32),
            pltpu.VMEM((IDX_PAD,), jnp.float32),
            pltpu.SemaphoreType.DMA,
            pltpu.SemaphoreType.DMA,
        ],
    )


def _loss_body(temp_ref, pos_ref, neg_ref, out_ref):
    inv_t = 1.0 / temp_ref[0]
    posl = pos_ref[...] * inv_t
    x = neg_ref[...] * inv_t
    m = jnp.maximum(jnp.max(x, axis=1, keepdims=True), posl)
    s = jnp.sum(jnp.exp(x - m), axis=1) + jnp.exp(posl - m)[:, 0]
    lse = m[:, 0] + jnp.log(s)
    contrib = jnp.sum(lse - posl[:, 0]) * (1.0 / B)
    out_ref[...] = jnp.full((1, 1), contrib, dtype=jnp.float32)


_loss = pl.pallas_call(
    _loss_body,
    grid=(1,),
    in_specs=[
        pl.BlockSpec(memory_space=pltpu.SMEM),
        pl.BlockSpec((B, 1), lambda i: (0, 0)),
        pl.BlockSpec((B, K), lambda i: (0, 0)),
    ],
    out_specs=pl.BlockSpec((1, 1), lambda i: (0, 0)),
    out_shape=jax.ShapeDtypeStruct((1, 1), jnp.float32),
)


def kernel(z_i, z_j, temperature, neg_indices):
    pool3, poolpk3, pos = _normalize(z_i, z_j)
    pool = pool3.reshape(2 * B, D)
    poolpk = poolpk3.reshape(2 * B, D // 2)
    neg_flat = _get_sc_negdot()(pool, poolpk, neg_indices.reshape(B * K))
    neg = neg_flat.reshape(B, K)
    out = _loss(temperature.reshape(1), pos, neg)
    return out[0, 0]


# speedup vs baseline: 1.5381x; 1.0298x over previous
import functools

import jax
import jax.numpy as jnp
from jax import lax
from jax.experimental import pallas as pl
from jax.experimental.pallas import tpu as pltpu
from jax.experimental.pallas import tpu_sc as plsc

B = 16384
D = 128
K = 30
KP = 32
EPS_NORM = 1e-12

NW = 32
ROWS_PW = B // NW
IDX_PW = ROWS_PW * KP
ROWS_SUPER = 64
SUPER = ROWS_PW // ROWS_SUPER
CHUNK_ROWS = 4
CHUNK_IDX = CHUNK_ROWS * KP
CHUNKS_PER_SUPER = ROWS_SUPER // CHUNK_ROWS
LANES = 16
NCHUNK16 = D // LANES

BB = 1024


def _normalize_body(zi_ref, zj_ref, idx_ref, out_ref, outpk_ref, pos_ref, idx32_ref):
    ys = []
    for s, ref in ((0, zi_ref), (1, zj_ref)):
        x = ref[...]
        n = jnp.sqrt(jnp.sum(x * x, axis=1, keepdims=True))
        y = x / jnp.maximum(n, EPS_NORM)
        ys.append(y)
        out_ref[s] = y
        yb = lax.bitcast_convert_type(y, jnp.int32)
        r = yb + jnp.int32(0x7FFF) + ((yb >> 16) & 1)
        a = r[:, : D // 2]
        b = r[:, D // 2 :]
        w = ((a >> 16) & jnp.int32(0xFFFF)) | (b & jnp.int32(-65536))
        outpk_ref[s] = w
    pos_ref[...] = jnp.sum(ys[0] * ys[1], axis=1, keepdims=True)
    idx = idx_ref[...]
    idx32_ref[...] = jnp.concatenate([idx, idx[:, :2]], axis=1)


_normalize = pl.pallas_call(
    _normalize_body,
    grid=(B // BB,),
    in_specs=[
        pl.BlockSpec((BB, D), lambda i: (i, 0)),
        pl.BlockSpec((BB, D), lambda i: (i, 0)),
        pl.BlockSpec((BB, K), lambda i: (i, 0)),
    ],
    out_specs=[
        pl.BlockSpec((2, BB, D), lambda i: (0, i, 0)),
        pl.BlockSpec((2, BB, D // 2), lambda i: (0, i, 0)),
        pl.BlockSpec((BB, 1), lambda i: (i, 0)),
        pl.BlockSpec((BB, KP), lambda i: (i, 0)),
    ],
    out_shape=[
        jax.ShapeDtypeStruct((2, B, D), jnp.float32),
        jax.ShapeDtypeStruct((2, B, D // 2), jnp.int32),
        jax.ShapeDtypeStruct((B, 1), jnp.float32),
        jax.ShapeDtypeStruct((B, KP), jnp.int32),
    ],
)


_GATHER_1D = lax.GatherDimensionNumbers(
    offset_dims=(), collapsed_slice_dims=(0,), start_index_map=(0,)
)


def _lane_perm(x, perm_2d):
    return lax.gather(
        x,
        perm_2d,
        dimension_numbers=_GATHER_1D,
        slice_sizes=(1,),
        mode=lax.GatherScatterMode.PROMISE_IN_BOUNDS,
    )


def _lane_sum_16(accs, perms, uppers):
    vecs = list(accs)
    for lev in range(4):
        perm = perms[lev]
        upper = uppers[lev]
        nxt = []
        for p in range(0, len(vecs), 2):
            ta = vecs[p] + _lane_perm(vecs[p], perm)
            tb = vecs[p + 1] + _lane_perm(vecs[p + 1], perm)
            nxt.append(jnp.where(upper, tb, ta))
        vecs = nxt
    return vecs[0]


NCHUNKS = IDX_PW // CHUNK_IDX


def _sc_negdot_body(
    pool_hbm, poolpk_hbm, idx_hbm, out_hbm, idx_v, zi_v, rows_v, out_v,
    sem0, sem1
):
    wid = lax.axis_index("s") * 2 + lax.axis_index("c")
    idx_base = wid * IDX_PW
    row_base = wid * ROWS_PW
    lane_iota = lax.iota(jnp.int32, LANES)
    perms = [(lane_iota ^ (1 << lev))[:, None] for lev in range(4)]
    uppers = [(lane_iota & (1 << lev)) != 0 for lev in range(4)]
    pltpu.sync_copy(idx_hbm.at[pl.ds(idx_base, IDX_PW)], idx_v)

    def fire(c):
        src = poolpk_hbm.at[idx_v.at[pl.ds(c * CHUNK_IDX, CHUNK_IDX)]]

        @pl.when((c & 1) == 0)
        def _():
            pltpu.async_copy(src, rows_v.at[0], sem0)

        @pl.when((c & 1) == 1)
        def _():
            pltpu.async_copy(src, rows_v.at[1], sem1)

    fire(0)

    def chunk_body(cc, carry):
        @pl.when(lax.rem(cc, CHUNKS_PER_SUPER) == 0)
        def _():
            pltpu.sync_copy(
                pool_hbm.at[
                    pl.ds(row_base + (cc // CHUNKS_PER_SUPER) * ROWS_SUPER,
                          ROWS_SUPER)
                ],
                zi_v,
            )

        @pl.when(cc + 1 < NCHUNKS)
        def _():
            fire(cc + 1)

        dummy = poolpk_hbm.at[pl.ds(0, CHUNK_IDX)]

        @pl.when((cc & 1) == 0)
        def _():
            pltpu.make_async_copy(dummy, rows_v.at[0], sem0).wait()

        @pl.when((cc & 1) == 1)
        def _():
            pltpu.make_async_copy(dummy, rows_v.at[1], sem1).wait()

        par = cc & 1
        off = cc * CHUNK_IDX
        rloc = lax.rem(cc, CHUNKS_PER_SUPER) * CHUNK_ROWS
        for j in range(CHUNK_ROWS):
            zi_regs = [
                zi_v[rloc + j, pl.ds(LANES * t, LANES)] for t in range(NCHUNK16)
            ]
            for grp in range(KP // LANES):
                accs = []
                for l in range(LANES):
                    g = j * KP + grp * LANES + l
                    acc = None
                    for t2 in range(D // 32):
                        w = rows_v[par, g, pl.ds(LANES * t2, LANES)]
                        e = lax.bitcast_convert_type(w << 16, jnp.float32)
                        o = lax.bitcast_convert_type(w, jnp.float32)
                        pa = e * zi_regs[t2]
                        pb = o * zi_regs[t2 + NCHUNK16 // 2]
                        acc = pa + pb if acc is None else acc + pa + pb
                    accs.append(acc)
                dots = _lane_sum_16(accs, perms, uppers)
                out_v[pl.ds(off + j * KP + grp * LANES, LANES)] = dots
        return carry

    lax.fori_loop(0, NCHUNKS, chunk_body, 0)
    pltpu.sync_copy(out_v, out_hbm.at[pl.ds(idx_base, IDX_PW)])


@functools.cache
def _get_sc_negdot():
    return pl.kernel(
        _sc_negdot_body,
        out_type=jax.ShapeDtypeStruct((B * KP,), jnp.float32),
        mesh=plsc.VectorSubcoreMesh(core_axis_name="c", subcore_axis_name="s"),
        compiler_params=pltpu.CompilerParams(use_tc_tiling_on_sc=False),
        scratch_types=[
            pltpu.VMEM((IDX_PW,), jnp.int32),
            pltpu.VMEM((ROWS_SUPER, D), jnp.float32),
            pltpu.VMEM((2, CHUNK_IDX, D // 2), jnp.int32),
            pltpu.VMEM((IDX_PW,), jnp.float32),
            pltpu.SemaphoreType.DMA,
            pltpu.SemaphoreType.DMA,
        ],
    )


def _loss_body(temp_ref, pos_ref, neg_ref, out_ref):
    inv_t = 1.0 / temp_ref[0]
    posl = pos_ref[...] * inv_t
    col = lax.broadcasted_iota(jnp.int32, (B, KP), 1)
    x = jnp.where(col < K, neg_ref[...] * inv_t, -3e38)
    m = jnp.maximum(jnp.max(x, axis=1, keepdims=True), posl)
    s = jnp.sum(jnp.exp(x - m), axis=1) + jnp.exp(posl - m)[:, 0]
    lse = m[:, 0] + jnp.log(s)
    contrib = jnp.sum(lse - posl[:, 0]) * (1.0 / B)
    out_ref[...] = jnp.full((1, 1), contrib, dtype=jnp.float32)


_loss = pl.pallas_call(
    _loss_body,
    grid=(1,),
    in_specs=[
        pl.BlockSpec(memory_space=pltpu.SMEM),
        pl.BlockSpec((B, 1), lambda i: (0, 0)),
        pl.BlockSpec((B, KP), lambda i: (0, 0)),
    ],
    out_specs=pl.BlockSpec((1, 1), lambda i: (0, 0)),
    out_shape=jax.ShapeDtypeStruct((1, 1), jnp.float32),
)


def kernel(z_i, z_j, temperature, neg_indices):
    pool3, poolpk3, pos, idx32 = _normalize(z_i, z_j, neg_indices)
    pool = pool3.reshape(2 * B, D)
    poolpk = poolpk3.reshape(2 * B, D // 2)
    neg_flat = _get_sc_negdot()(pool, poolpk, idx32.reshape(B * KP))
    neg = neg_flat.reshape(B, KP)
    out = _loss(temperature.reshape(1), pos, neg)
    return out[0, 0]


# speedup vs baseline: 1.5779x; 1.0258x over previous
import functools

import jax
import jax.numpy as jnp
from jax import lax
from jax.experimental import pallas as pl
from jax.experimental.pallas import tpu as pltpu
from jax.experimental.pallas import tpu_sc as plsc

B = 16384
D = 128
K = 30
KP = 32
EPS_NORM = 1e-12

NW = 32
ROWS_PW = B // NW
IDX_PW = ROWS_PW * KP
ROWS_SUPER = 64
SUPER = ROWS_PW // ROWS_SUPER
CHUNK_ROWS = 4
CHUNK_IDX = CHUNK_ROWS * KP
CHUNKS_PER_SUPER = ROWS_SUPER // CHUNK_ROWS
LANES = 16
NCHUNK16 = D // LANES

BB = 1024


def _normalize_body(zi_ref, zj_ref, out_ref, outpk_ref, pos_ref):
    ys = []
    for s, ref in ((0, zi_ref), (1, zj_ref)):
        x = ref[...]
        n = jnp.sqrt(jnp.sum(x * x, axis=1, keepdims=True))
        y = x / jnp.maximum(n, EPS_NORM)
        ys.append(y)
        out_ref[s] = y
        yb = lax.bitcast_convert_type(y, jnp.int32)
        r = yb + jnp.int32(0x7FFF) + ((yb >> 16) & 1)
        a = r[:, : D // 2]
        b = r[:, D // 2 :]
        w = ((a >> 16) & jnp.int32(0xFFFF)) | (b & jnp.int32(-65536))
        outpk_ref[s] = w
    pos_ref[...] = jnp.sum(ys[0] * ys[1], axis=1, keepdims=True)


_normalize = pl.pallas_call(
    _normalize_body,
    grid=(B // BB,),
    in_specs=[
        pl.BlockSpec((BB, D), lambda i: (i, 0)),
        pl.BlockSpec((BB, D), lambda i: (i, 0)),
    ],
    out_specs=[
        pl.BlockSpec((2, BB, D), lambda i: (0, i, 0)),
        pl.BlockSpec((2, BB, D // 2), lambda i: (0, i, 0)),
        pl.BlockSpec((BB, 1), lambda i: (i, 0)),
    ],
    out_shape=[
        jax.ShapeDtypeStruct((2, B, D), jnp.float32),
        jax.ShapeDtypeStruct((2, B, D // 2), jnp.int32),
        jax.ShapeDtypeStruct((B, 1), jnp.float32),
    ],
)


_GATHER_1D = lax.GatherDimensionNumbers(
    offset_dims=(), collapsed_slice_dims=(0,), start_index_map=(0,)
)


def _lane_perm(x, perm_2d):
    return lax.gather(
        x,
        perm_2d,
        dimension_numbers=_GATHER_1D,
        slice_sizes=(1,),
        mode=lax.GatherScatterMode.PROMISE_IN_BOUNDS,
    )


def _lane_sum_16(accs, perms, uppers):
    vecs = list(accs)
    for lev in range(4):
        perm = perms[lev]
        upper = uppers[lev]
        nxt = []
        for p in range(0, len(vecs), 2):
            ta = vecs[p] + _lane_perm(vecs[p], perm)
            tb = vecs[p + 1] + _lane_perm(vecs[p + 1], perm)
            nxt.append(jnp.where(upper, tb, ta))
        vecs = nxt
    return vecs[0]


NCHUNKS = IDX_PW // CHUNK_IDX


def _sc_negdot_body(
    pool_hbm, poolpk_hbm, idx_hbm, out_hbm, idx_v, zi_v, rows_v, out_v,
    sem0, sem1
):
    wid = lax.axis_index("s") * 2 + lax.axis_index("c")
    idx_base = wid * IDX_PW
    row_base = wid * ROWS_PW
    lane_iota = lax.iota(jnp.int32, LANES)
    perms = [(lane_iota ^ (1 << lev))[:, None] for lev in range(4)]
    uppers = [(lane_iota & (1 << lev)) != 0 for lev in range(4)]
    pltpu.sync_copy(idx_hbm.at[pl.ds(idx_base, IDX_PW)], idx_v)

    def fire(c):
        src = poolpk_hbm.at[idx_v.at[pl.ds(c * CHUNK_IDX, CHUNK_IDX)]]

        @pl.when((c & 1) == 0)
        def _():
            pltpu.async_copy(src, rows_v.at[0], sem0)

        @pl.when((c & 1) == 1)
        def _():
            pltpu.async_copy(src, rows_v.at[1], sem1)

    fire(0)

    def chunk_body(cc, carry):
        @pl.when(lax.rem(cc, CHUNKS_PER_SUPER) == 0)
        def _():
            pltpu.sync_copy(
                pool_hbm.at[
                    pl.ds(row_base + (cc // CHUNKS_PER_SUPER) * ROWS_SUPER,
                          ROWS_SUPER)
                ],
                zi_v,
            )

        @pl.when(cc + 1 < NCHUNKS)
        def _():
            fire(cc + 1)

        dummy = poolpk_hbm.at[pl.ds(0, CHUNK_IDX)]

        @pl.when((cc & 1) == 0)
        def _():
            pltpu.make_async_copy(dummy, rows_v.at[0], sem0).wait()

        @pl.when((cc & 1) == 1)
        def _():
            pltpu.make_async_copy(dummy, rows_v.at[1], sem1).wait()

        par = cc & 1
        off = cc * CHUNK_IDX
        rloc = lax.rem(cc, CHUNKS_PER_SUPER) * CHUNK_ROWS
        for j in range(CHUNK_ROWS):
            zi_regs = [
                zi_v[rloc + j, pl.ds(LANES * t, LANES)] for t in range(NCHUNK16)
            ]
            for grp in range(KP // LANES):
                accs = []
                for l in range(LANES):
                    g = j * KP + grp * LANES + l
                    chains = [None, None]
                    for t2 in range(D // 32):
                        w = rows_v[par, g, pl.ds(LANES * t2, LANES)]
                        e = lax.bitcast_convert_type(w << 16, jnp.float32)
                        o = lax.bitcast_convert_type(w, jnp.float32)
                        pa = e * zi_regs[t2]
                        pb = o * zi_regs[t2 + NCHUNK16 // 2]
                        c = t2 & 1
                        chains[c] = (
                            pa + pb if chains[c] is None
                            else chains[c] + pa + pb
                        )
                    accs.append(chains[0] + chains[1])
                dots = _lane_sum_16(accs, perms, uppers)
                out_v[pl.ds(off + j * KP + grp * LANES, LANES)] = dots
        return carry

    lax.fori_loop(0, NCHUNKS, chunk_body, 0)
    pltpu.sync_copy(out_v, out_hbm.at[pl.ds(idx_base, IDX_PW)])


@functools.cache
def _get_sc_negdot():
    return pl.kernel(
        _sc_negdot_body,
        out_type=jax.ShapeDtypeStruct((B * KP,), jnp.float32),
        mesh=plsc.VectorSubcoreMesh(core_axis_name="c", subcore_axis_name="s"),
        compiler_params=pltpu.CompilerParams(use_tc_tiling_on_sc=False),
        scratch_types=[
            pltpu.VMEM((IDX_PW,), jnp.int32),
            pltpu.VMEM((ROWS_SUPER, D), jnp.float32),
            pltpu.VMEM((2, CHUNK_IDX, D // 2), jnp.int32),
            pltpu.VMEM((IDX_PW,), jnp.float32),
            pltpu.SemaphoreType.DMA,
            pltpu.SemaphoreType.DMA,
        ],
    )


def _loss_body(temp_ref, pos_ref, neg_ref, out_ref):
    inv_t = 1.0 / temp_ref[0]
    posl = pos_ref[...] * inv_t
    col = lax.broadcasted_iota(jnp.int32, (B, KP), 1)
    x = jnp.where(col < K, neg_ref[...] * inv_t, -3e38)
    m = jnp.maximum(jnp.max(x, axis=1, keepdims=True), posl)
    s = jnp.sum(jnp.exp(x - m), axis=1) + jnp.exp(posl - m)[:, 0]
    lse = m[:, 0] + jnp.log(s)
    contrib = jnp.sum(lse - posl[:, 0]) * (1.0 / B)
    out_ref[...] = jnp.full((1, 1), contrib, dtype=jnp.float32)


_loss = pl.pallas_call(
    _loss_body,
    grid=(1,),
    in_specs=[
        pl.BlockSpec(memory_space=pltpu.SMEM),
        pl.BlockSpec((B, 1), lambda i: (0, 0)),
        pl.BlockSpec((B, KP), lambda i: (0, 0)),
    ],
    out_specs=pl.BlockSpec((1, 1), lambda i: (0, 0)),
    out_shape=jax.ShapeDtypeStruct((1, 1), jnp.float32),
)


def kernel(z_i, z_j, temperature, neg_indices):
    pool3, poolpk3, pos = _normalize(z_i, z_j)
    pool = pool3.reshape(2 * B, D)
    poolpk = poolpk3.reshape(2 * B, D // 2)
    idx32 = jnp.concatenate([neg_indices, neg_indices[:, :2]], axis=1)
    neg_flat = _get_sc_negdot()(pool, poolpk, idx32.reshape(B * KP))
    neg = neg_flat.reshape(B, KP)
    out = _loss(temperature.reshape(1), pos, neg)
    return out[0, 0]
